# baseline (speedup 1.0000x reference)
import functools

import jax
import jax.numpy as jnp
from jax import lax
from jax.experimental import pallas as pl
from jax.experimental.pallas import tpu as pltpu
from jax.experimental.pallas import tpu_sc as plsc

_B = 16384
_F = 26
_V = 2
_D = 128

_NC = 2
_NS = 16
_NW = _NC * _NS

_REPS = 256
_N = _B * _F
_PER_W = _N // _NW
_CH = 128
_NCH = _PER_W // _CH
_NBUF = 4
_KIND = (0, 0, 1, 1)


def _lookup(xf, table2, tabflat):
    mesh = plsc.VectorSubcoreMesh(core_axis_name="c", subcore_axis_name="s")

    @functools.partial(
        pl.kernel,
        out_type=jax.ShapeDtypeStruct((_N, _D), jnp.float32),
        mesh=mesh,
        scratch_types=[
            pltpu.VMEM((_PER_W,), jnp.int32),
            pltpu.VMEM((_F * _V * _D,), jnp.float32),
            pltpu.VMEM((_NBUF, _CH), jnp.int32),
            pltpu.VMEM((_NBUF, _CH, _D), jnp.float32),
            pltpu.SemaphoreType.DMA,
            pltpu.SemaphoreType.DMA,
            pltpu.SemaphoreType.DMA,
            pltpu.SemaphoreType.DMA,
            pltpu.SemaphoreType.DMA,
            pltpu.SemaphoreType.DMA,
            pltpu.SemaphoreType.DMA,
            pltpu.SemaphoreType.DMA,
        ],
    )
    def body(xf_hbm, tab_hbm, tabf_hbm, out_hbm, xall, tabv, idxs, rows,
             g0, g1, g2, g3, s0, s1, s2, s3):
        gsem = (g0, g1, g2, g3)
        osem = (s0, s1, s2, s3)
        wid = lax.axis_index("s") * _NC + lax.axis_index("c")
        wbase = wid * _PER_W
        pltpu.sync_copy(tabf_hbm, tabv)
        pltpu.sync_copy(xf_hbm.at[pl.ds(wbase, _PER_W)], xall)

        lanes = lax.iota(jnp.int32, 16)

        def compute_idx(j, b):
            base = j * _CH
            for g in range(_CH // 16):
                off = base + g * 16
                pos = (wbase + off) + lanes
                f = lax.rem(pos, _F)
                rep = lax.bitwise_and(pos, _REPS - 1) * (_F * _V)
                idxs[b, pl.ds(g * 16, 16)] = (
                    xall[pl.ds(off, 16)] + 2 * f + rep)

        def build_chunk(j, b):
            base = j * _CH

            def group(g, carry):
                off = base + g * 16
                pos = (wbase + off) + lanes
                f = lax.rem(pos, _F)
                iv = (xall[pl.ds(off, 16)] + 2 * f) * _D
                r0 = g * 16
                for l in range(16):
                    roff = iv[l]
                    for c in range(_D // 16):
                        rows[b, r0 + l, pl.ds(c * 16, 16)] = (
                            tabv[pl.ds(roff + c * 16, 16)])
                return carry

            lax.fori_loop(0, _CH // 16, group, 0)

        def fire_gather(b):
            pltpu.async_copy(tab_hbm.at[idxs.at[b]], rows.at[b], gsem[b])

        def wait_gather(b):
            pltpu.make_async_copy(
                tab_hbm.at[idxs.at[b]], rows.at[b], gsem[b]).wait()

        def fire_scatter(j, b):
            pltpu.async_copy(
                rows.at[b], out_hbm.at[pl.ds(wbase + j * _CH, _CH)], osem[b])

        def wait_scatter(b):
            pltpu.make_async_copy(
                rows.at[b], out_hbm.at[pl.ds(wbase, _CH)], osem[b]).wait()

        def work(j, b, kind):
            if kind == 0:
                wait_gather(b)
            else:
                build_chunk(j, b)
            fire_scatter(j, b)

        def prep(jg, bg, kind, first):
            if kind == 0:
                if not first:
                    wait_scatter(bg)
                compute_idx(jg, bg)
                fire_gather(bg)
            elif not first:
                wait_scatter(bg)

        prep(0, 0, _KIND[0], True)
        prep(1, 1, _KIND[1], True)
        work(0, 0, _KIND[0])
        prep(2, 2, _KIND[2], True)
        work(1, 1, _KIND[1])
        prep(3, 3, _KIND[3], True)

        def outer(s, carry):
            for k in range(_NBUF):
                j = 2 + s * _NBUF + k
                b = (2 + k) % _NBUF
                bg = k % _NBUF
                work(j, b, _KIND[(2 + k) % _NBUF])
                prep(j + 2, bg, _KIND[k % _NBUF], False)
            return carry

        lax.fori_loop(0, (_NCH - 4) // _NBUF, outer, 0)

        for j in range(_NCH - 2, _NCH):
            work(j, j % _NBUF, _KIND[j % _NBUF])
        for b in range(_NBUF):
            wait_scatter(b)

    return body(xf, table2, tabflat)


def kernel(X, tables):
    xf = X.reshape(_N)
    table2 = jnp.tile(tables.reshape(_F * _V, _D), (_REPS, 1))
    tabflat = tables.reshape(_F * _V * _D)
    out = _lookup(xf, table2, tabflat)
    return out.reshape(_B, _F, 1, _D)
